# Initial kernel scaffold; baseline (speedup 1.0000x reference)
import jax
import jax.numpy as jnp
from jax.experimental import pallas as pl


def kernel(pose1, pose2, connections, W1, att_src1, att_dst1, b1, W2, att_src2, att_dst2, b2, Wfe, bfe, bn_gamma, bn_beta, bn_mean, bn_var, Wih_f, Whh_f, bih_f, bhh_f, Wih_b, Whh_b, bih_b, bhh_b, Watt, batt, Wcls, bcls):
    raise NotImplementedError("write your pallas kernel here")


# speedup vs baseline: 5.3371x; 5.3371x over previous
import jax
import jax.numpy as jnp
from jax.experimental import pallas as pl
from jax.experimental.pallas import tpu as pltpu

B, T, V, E = 32, 12, 17, 38
HEADS, HC, HL, NCLS = 8, 128, 512, 500
G = 2 * T
NGV = G * V
NGE = G * E
HID = HEADS * HC
D = HC * (V + 2)
F32 = jnp.float32
HI = jax.lax.Precision.HIGHEST


def _gat_body(x_ref, sels_ref, selt_ref, seltt_ref, w1_ref, as1_ref, ad1_ref,
              b1_ref, w2_ref, as2_ref, ad2_ref, b2_ref, sum8_ref, rep8_ref,
              avg_ref, out_ref):
    def layer(h, asrc, adst):
        als = jnp.dot(h * asrc, sum8_ref[...], preferred_element_type=F32, precision=HI)
        ald = jnp.dot(h * adst, sum8_ref[...], preferred_element_type=F32, precision=HI)
        al = (jnp.dot(sels_ref[...], als, preferred_element_type=F32, precision=HI)
              + jnp.dot(selt_ref[...], ald, preferred_element_type=F32, precision=HI))
        al = jnp.where(al >= 0.0, al, 0.2 * al)
        m = jnp.max(al, axis=0, keepdims=True)
        e = jnp.exp(al - m)
        den = jnp.dot(seltt_ref[...], e, preferred_element_type=F32, precision=HI)
        den_e = jnp.dot(selt_ref[...], den, preferred_element_type=F32, precision=HI)
        a = e / (den_e + 1e-16)
        hs = jnp.dot(sels_ref[...], h, preferred_element_type=F32, precision=HI)
        aexp = jnp.dot(a, rep8_ref[...], preferred_element_type=F32, precision=HI)
        return jnp.dot(seltt_ref[...], hs * aexp, preferred_element_type=F32, precision=HI)

    h1 = jnp.dot(x_ref[...], w1_ref[...], preferred_element_type=F32, precision=HI)
    o1 = layer(h1, as1_ref[...], ad1_ref[...]) + b1_ref[...]
    x1 = jnp.where(o1 > 0.0, o1, jnp.exp(jnp.minimum(o1, 0.0)) - 1.0)
    h2 = jnp.dot(x1, w2_ref[...], preferred_element_type=F32, precision=HI)
    o2 = layer(h2, as2_ref[...], ad2_ref[...])
    out_ref[...] = jnp.dot(o2, avg_ref[...], preferred_element_type=F32, precision=HI) + b2_ref[...]


def _edge_body(px_ref, py_ref, d0_ref, d1_ref, wa_ref, wb_ref, bfe_ref, out_ref):
    px, py = px_ref[...], py_ref[...]
    for r, d_ref in ((0, d0_ref), (1, d1_ref)):
        vx = jnp.dot(px, d_ref[...], preferred_element_type=F32, precision=HI)
        vy = jnp.dot(py, d_ref[...], preferred_element_type=F32, precision=HI)
        ln = jnp.sqrt(vx * vx + vy * vy)
        ang = jnp.arctan2(vy, vx)
        o = (jnp.dot(ln, wa_ref[...], preferred_element_type=F32, precision=HI)
             + jnp.dot(ang, wb_ref[...], preferred_element_type=F32, precision=HI)
             + bfe_ref[...])
        out_ref[:, r * HC:(r + 1) * HC] = o


def _proj_body(x_ref, scv_ref, shv_ref, wt_ref, b_ref, out_ref):
    xn = x_ref[...] * scv_ref[...] + shv_ref[...]
    out_ref[...] = jnp.dot(xn, wt_ref[...], preferred_element_type=F32, precision=HI) + b_ref[0]


def _lstm_body(gf_ref, gb_ref, whf_ref, whb_ref, watt_ref, wcls_ref, bcls_ref,
               att_ref, cls_ref, lo_ref):
    nb = 2 * B

    def cell(g):
        i = jax.nn.sigmoid(g[:, 0:HL])
        f = jax.nn.sigmoid(g[:, HL:2 * HL])
        gg = jnp.tanh(g[:, 2 * HL:3 * HL])
        o = jax.nn.sigmoid(g[:, 3 * HL:4 * HL])
        return i, f, gg, o

    hf = jnp.zeros((nb, HL), F32)
    cf = jnp.zeros((nb, HL), F32)
    hb = jnp.zeros((nb, HL), F32)
    cb = jnp.zeros((nb, HL), F32)
    for t in range(T):
        gf = gf_ref[t] + jnp.dot(hf, whf_ref[...], preferred_element_type=F32, precision=HI)
        i, f, gg, o = cell(gf)
        cf = f * cf + i * gg
        hf = o * jnp.tanh(cf)
        lo_ref[t, :, 0:HL] = hf
        tb = T - 1 - t
        gb = gb_ref[tb] + jnp.dot(hb, whb_ref[...], preferred_element_type=F32, precision=HI)
        i, f, gg, o = cell(gb)
        cb = f * cb + i * gg
        hb = o * jnp.tanh(cb)
        lo_ref[tb, :, HL:2 * HL] = hb

    scores = jnp.concatenate(
        [jnp.dot(lo_ref[t], watt_ref[...], preferred_element_type=F32, precision=HI)
         for t in range(T)], axis=1)
    m = jnp.max(scores, axis=1, keepdims=True)
    e = jnp.exp(scores - m)
    aw = e / jnp.sum(e, axis=1, keepdims=True)
    att = jnp.zeros((nb, 2 * HL), F32)
    for t in range(T):
        att = att + aw[:, t:t + 1] * lo_ref[t]
    att_ref[...] = att
    cls_ref[...] = jnp.dot(att, wcls_ref[...], preferred_element_type=F32, precision=HI) + bcls_ref[...]


def kernel(pose1, pose2, connections, W1, att_src1, att_dst1, b1, W2, att_src2,
           att_dst2, b2, Wfe, bfe, bn_gamma, bn_beta, bn_mean, bn_var, Wih_f,
           Whh_f, bih_f, bhh_f, Wih_b, Whh_b, bih_b, bhh_b, Watt, batt, Wcls,
           bcls):
    s_idx = connections[0].astype(jnp.int32)
    t_idx = connections[1].astype(jnp.int32)

    gofs = jnp.arange(G, dtype=jnp.int32) * V
    sel_s = jax.nn.one_hot((gofs[:, None] + s_idx[None, :]).reshape(NGE), NGV, dtype=F32)
    sel_t = jax.nn.one_hot((gofs[:, None] + t_idx[None, :]).reshape(NGE), NGV, dtype=F32)
    sel_tt = sel_t.T
    sum8 = jax.nn.one_hot(jnp.arange(HID, dtype=jnp.int32) // HC, HEADS, dtype=F32)
    rep8 = sum8.T
    avg8 = jax.nn.one_hot(jnp.arange(HID, dtype=jnp.int32) % HC, HC, dtype=F32) / HEADS

    x24 = jnp.concatenate([pose1[0], pose2[0]], axis=0).reshape(NGV, 3)
    gat_nodes = pl.pallas_call(
        _gat_body,
        out_shape=jax.ShapeDtypeStruct((NGV, HC), F32),
    )(x24, sel_s, sel_t, sel_tt, W1, att_src1.reshape(1, HID),
      att_dst1.reshape(1, HID), b1.reshape(1, HID), W2,
      att_src2.reshape(1, HID), att_dst2.reshape(1, HID), b2.reshape(1, HC),
      sum8, rep8, avg8)

    pall = jnp.concatenate([pose1, pose2], axis=0).reshape(2 * B * T, V, 3)
    px, py = pall[:, :, 0], pall[:, :, 1]
    dmat = (jax.nn.one_hot(t_idx, V, dtype=F32) - jax.nn.one_hot(s_idx, V, dtype=F32)).T
    edge_out = pl.pallas_call(
        _edge_body,
        out_shape=jax.ShapeDtypeStruct((2 * B * T, 2 * HC), F32),
    )(px, py, dmat[:, :E // 2], dmat[:, E // 2:], Wfe[0::2], Wfe[1::2],
      bfe.reshape(1, HC))

    gat2 = gat_nodes.reshape(G, V * HC)
    dead = jnp.tile(b2, V)
    gat_part = jnp.broadcast_to(dead, (2 * B, T, V * HC))
    gat_part = gat_part.at[0].set(gat2[:T]).at[B].set(gat2[T:])
    comb = jnp.concatenate(
        [gat_part, edge_out.reshape(2 * B, T, 2 * HC)], axis=-1).reshape(2 * B * T, D)

    sc = bn_gamma / jnp.sqrt(bn_var + 1e-5)
    sh = bn_beta - bn_mean * sc
    scv = jnp.tile(sc, 2 * B).reshape(2 * B * T, 1)
    shv = jnp.tile(sh, 2 * B).reshape(2 * B * T, 1)
    wt = jnp.concatenate([Wih_f, Wih_b], axis=0).T
    bias = jnp.concatenate([bih_f + bhh_f, bih_b + bhh_b]).reshape(8, 1, HL)
    nblk = 8
    proj = pl.pallas_call(
        _proj_body,
        grid=(nblk,),
        in_specs=[
            pl.BlockSpec((2 * B * T, D), lambda i: (0, 0)),
            pl.BlockSpec((2 * B * T, 1), lambda i: (0, 0)),
            pl.BlockSpec((2 * B * T, 1), lambda i: (0, 0)),
            pl.BlockSpec((D, HL), lambda i: (0, i)),
            pl.BlockSpec((1, 1, HL), lambda i: (i, 0, 0)),
        ],
        out_specs=pl.BlockSpec((2 * B * T, HL), lambda i: (0, i)),
        out_shape=jax.ShapeDtypeStruct((2 * B * T, 8 * HL), F32),
    )(comb, scv, shv, wt, bias)

    gf = proj[:, :4 * HL].reshape(2 * B, T, 4 * HL).transpose(1, 0, 2)
    gb = proj[:, 4 * HL:].reshape(2 * B, T, 4 * HL).transpose(1, 0, 2)

    att, cls = pl.pallas_call(
        _lstm_body,
        out_shape=(jax.ShapeDtypeStruct((2 * B, 2 * HL), F32),
                   jax.ShapeDtypeStruct((2 * B, NCLS), F32)),
        scratch_shapes=[pltpu.VMEM((T, 2 * B, 2 * HL), F32)],
    )(gf, gb, Whh_f.T, Whh_b.T, Watt, Wcls, bcls.reshape(1, NCLS))
    return att, cls


# speedup vs baseline: 6.8808x; 1.2893x over previous
import jax
import jax.numpy as jnp
from jax.experimental import pallas as pl
from jax.experimental.pallas import tpu as pltpu

B, T, V, E = 32, 12, 17, 38
HEADS, HC, HL, NCLS = 8, 128, 512, 500
G = 2 * T
NGV = G * V
NGE = G * E
HID = HEADS * HC
D = HC * (V + 2)
F32 = jnp.float32
HI = jax.lax.Precision.HIGHEST


def _gat_body(x_ref, sels_ref, selt_ref, seltt_ref, w1_ref, as1_ref, ad1_ref,
              b1_ref, w2_ref, as2_ref, ad2_ref, b2_ref, sum8_ref, rep8_ref,
              avg_ref, out_ref):
    def layer(h, asrc, adst):
        als = jnp.dot(h * asrc, sum8_ref[...], preferred_element_type=F32, precision=HI)
        ald = jnp.dot(h * adst, sum8_ref[...], preferred_element_type=F32, precision=HI)
        al = (jnp.dot(sels_ref[...], als, preferred_element_type=F32, precision=HI)
              + jnp.dot(selt_ref[...], ald, preferred_element_type=F32, precision=HI))
        al = jnp.where(al >= 0.0, al, 0.2 * al)
        m = jnp.max(al, axis=0, keepdims=True)
        e = jnp.exp(al - m)
        den = jnp.dot(seltt_ref[...], e, preferred_element_type=F32, precision=HI)
        den_e = jnp.dot(selt_ref[...], den, preferred_element_type=F32, precision=HI)
        a = e / (den_e + 1e-16)
        hs = jnp.dot(sels_ref[...], h, preferred_element_type=F32, precision=HI)
        aexp = jnp.dot(a, rep8_ref[...], preferred_element_type=F32, precision=HI)
        return jnp.dot(seltt_ref[...], hs * aexp, preferred_element_type=F32, precision=HI)

    h1 = jnp.dot(x_ref[...], w1_ref[...], preferred_element_type=F32, precision=HI)
    o1 = layer(h1, as1_ref[...], ad1_ref[...]) + b1_ref[...]
    x1 = jnp.where(o1 > 0.0, o1, jnp.exp(jnp.minimum(o1, 0.0)) - 1.0)
    h2 = jnp.dot(x1, w2_ref[...], preferred_element_type=F32, precision=HI)
    o2 = layer(h2, as2_ref[...], ad2_ref[...])
    out_ref[...] = jnp.dot(o2, avg_ref[...], preferred_element_type=F32, precision=HI) + b2_ref[...]


def _edge_body(px_ref, py_ref, d0_ref, d1_ref, wa_ref, wb_ref, bfe_ref, out_ref):
    px, py = px_ref[...], py_ref[...]
    for r, d_ref in ((0, d0_ref), (1, d1_ref)):
        vx = jnp.dot(px, d_ref[...], preferred_element_type=F32, precision=HI)
        vy = jnp.dot(py, d_ref[...], preferred_element_type=F32, precision=HI)
        ln = jnp.sqrt(vx * vx + vy * vy)
        ang = jnp.arctan2(vy, vx)
        o = (jnp.dot(ln, wa_ref[...], preferred_element_type=F32, precision=HI)
             + jnp.dot(ang, wb_ref[...], preferred_element_type=F32, precision=HI)
             + bfe_ref[...])
        out_ref[:, r * HC:(r + 1) * HC] = o


def _proj_body(xe_ref, scv_ref, shv_ref, xg_ref, scg_ref, shg_ref, sel_ref,
               wte_ref, wtg_ref, b_ref, out_ref):
    xg = xg_ref[...] * scg_ref[...] + shg_ref[...]
    g36 = jnp.dot(xg, wtg_ref[...], preferred_element_type=F32, precision=HI)
    xe = xe_ref[...] * scv_ref[...] + shv_ref[...]
    out_ref[...] = (jnp.dot(xe, wte_ref[...], preferred_element_type=F32, precision=HI)
                    + jnp.dot(sel_ref[...], g36, preferred_element_type=F32, precision=HI)
                    + b_ref[0])


def _lstm_body(g_ref, whf_ref, whb_ref, watt_ref, wcls_ref, bcls_ref,
               att_ref, cls_ref, lo_ref):
    nb = 2 * B

    def cell(g):
        i = jax.nn.sigmoid(g[:, 0:HL])
        f = jax.nn.sigmoid(g[:, HL:2 * HL])
        gg = jnp.tanh(g[:, 2 * HL:3 * HL])
        o = jax.nn.sigmoid(g[:, 3 * HL:4 * HL])
        return i, f, gg, o

    hf = jnp.zeros((nb, HL), F32)
    cf = jnp.zeros((nb, HL), F32)
    hb = jnp.zeros((nb, HL), F32)
    cb = jnp.zeros((nb, HL), F32)
    for t in range(T):
        gf = g_ref[t, :, 0:4 * HL] + jnp.dot(hf, whf_ref[...], preferred_element_type=F32, precision=HI)
        i, f, gg, o = cell(gf)
        cf = f * cf + i * gg
        hf = o * jnp.tanh(cf)
        lo_ref[t, :, 0:HL] = hf
        tb = T - 1 - t
        gb = g_ref[tb, :, 4 * HL:8 * HL] + jnp.dot(hb, whb_ref[...], preferred_element_type=F32, precision=HI)
        i, f, gg, o = cell(gb)
        cb = f * cb + i * gg
        hb = o * jnp.tanh(cb)
        lo_ref[tb, :, HL:2 * HL] = hb

    scores = jnp.concatenate(
        [jnp.dot(lo_ref[t], watt_ref[...], preferred_element_type=F32, precision=HI)
         for t in range(T)], axis=1)
    m = jnp.max(scores, axis=1, keepdims=True)
    e = jnp.exp(scores - m)
    aw = e / jnp.sum(e, axis=1, keepdims=True)
    att = jnp.zeros((nb, 2 * HL), F32)
    for t in range(T):
        att = att + aw[:, t:t + 1] * lo_ref[t]
    att_ref[...] = att
    cls_ref[...] = jnp.dot(att, wcls_ref[...], preferred_element_type=F32, precision=HI) + bcls_ref[...]


def kernel(pose1, pose2, connections, W1, att_src1, att_dst1, b1, W2, att_src2,
           att_dst2, b2, Wfe, bfe, bn_gamma, bn_beta, bn_mean, bn_var, Wih_f,
           Whh_f, bih_f, bhh_f, Wih_b, Whh_b, bih_b, bhh_b, Watt, batt, Wcls,
           bcls):
    s_idx = connections[0].astype(jnp.int32)
    t_idx = connections[1].astype(jnp.int32)

    gofs = jnp.arange(G, dtype=jnp.int32) * V
    sel_s = jax.nn.one_hot((gofs[:, None] + s_idx[None, :]).reshape(NGE), NGV, dtype=F32)
    sel_t = jax.nn.one_hot((gofs[:, None] + t_idx[None, :]).reshape(NGE), NGV, dtype=F32)
    sel_tt = sel_t.T
    sum8 = jax.nn.one_hot(jnp.arange(HID, dtype=jnp.int32) // HC, HEADS, dtype=F32)
    rep8 = sum8.T
    avg8 = jax.nn.one_hot(jnp.arange(HID, dtype=jnp.int32) % HC, HC, dtype=F32) / HEADS

    x24 = jnp.concatenate([pose1[0], pose2[0]], axis=0).reshape(NGV, 3)
    gat_nodes = pl.pallas_call(
        _gat_body,
        out_shape=jax.ShapeDtypeStruct((NGV, HC), F32),
    )(x24, sel_s, sel_t, sel_tt, W1, att_src1.reshape(1, HID),
      att_dst1.reshape(1, HID), b1.reshape(1, HID), W2,
      att_src2.reshape(1, HID), att_dst2.reshape(1, HID), b2.reshape(1, HC),
      sum8, rep8, avg8)

    pall = jnp.concatenate([pose1, pose2], axis=0).transpose(1, 0, 2, 3)
    pall = pall.reshape(2 * B * T, V, 3)
    px, py = pall[:, :, 0], pall[:, :, 1]
    dmat = (jax.nn.one_hot(t_idx, V, dtype=F32) - jax.nn.one_hot(s_idx, V, dtype=F32)).T
    edge_out = pl.pallas_call(
        _edge_body,
        out_shape=jax.ShapeDtypeStruct((2 * B * T, 2 * HC), F32),
    )(px, py, dmat[:, :E // 2], dmat[:, E // 2:], Wfe[0::2], Wfe[1::2],
      bfe.reshape(1, HC))

    sc = bn_gamma / jnp.sqrt(bn_var + 1e-5)
    sh = bn_beta - bn_mean * sc
    gat2 = gat_nodes.reshape(G, V * HC)
    dead = jnp.tile(b2, V)
    xg36 = jnp.concatenate(
        [jnp.broadcast_to(dead, (T, V * HC)), gat2], axis=0)
    scg = jnp.tile(sc, 3).reshape(3 * T, 1)
    shg = jnp.tile(sh, 3).reshape(3 * T, 1)
    tcol = jnp.arange(2 * B * T, dtype=jnp.int32) // (2 * B)
    bcol = jnp.arange(2 * B * T, dtype=jnp.int32) % (2 * B)
    sel_idx = jnp.where(bcol == 0, 12 + tcol,
                        jnp.where(bcol == B, 12 + T + tcol, tcol))
    sel768 = jax.nn.one_hot(sel_idx, 3 * T, dtype=F32)

    scv = jnp.repeat(sc, 2 * B).reshape(2 * B * T, 1)
    shv = jnp.repeat(sh, 2 * B).reshape(2 * B * T, 1)
    wt = jnp.concatenate([Wih_f, Wih_b], axis=0).T
    wte = wt[V * HC:]
    wtg = wt[:V * HC]
    bias = jnp.concatenate([bih_f + bhh_f, bih_b + bhh_b]).reshape(8, 1, HL)
    nblk = 8
    proj = pl.pallas_call(
        _proj_body,
        grid=(nblk,),
        in_specs=[
            pl.BlockSpec((2 * B * T, 2 * HC), lambda i: (0, 0)),
            pl.BlockSpec((2 * B * T, 1), lambda i: (0, 0)),
            pl.BlockSpec((2 * B * T, 1), lambda i: (0, 0)),
            pl.BlockSpec((3 * T, V * HC), lambda i: (0, 0)),
            pl.BlockSpec((3 * T, 1), lambda i: (0, 0)),
            pl.BlockSpec((3 * T, 1), lambda i: (0, 0)),
            pl.BlockSpec((2 * B * T, 3 * T), lambda i: (0, 0)),
            pl.BlockSpec((2 * HC, HL), lambda i: (0, i)),
            pl.BlockSpec((V * HC, HL), lambda i: (0, i)),
            pl.BlockSpec((1, 1, HL), lambda i: (i, 0, 0)),
        ],
        out_specs=pl.BlockSpec((2 * B * T, HL), lambda i: (0, i)),
        out_shape=jax.ShapeDtypeStruct((2 * B * T, 8 * HL), F32),
    )(edge_out, scv, shv, xg36, scg, shg, sel768, wte, wtg, bias)

    att, cls = pl.pallas_call(
        _lstm_body,
        out_shape=(jax.ShapeDtypeStruct((2 * B, 2 * HL), F32),
                   jax.ShapeDtypeStruct((2 * B, NCLS), F32)),
        scratch_shapes=[pltpu.VMEM((T, 2 * B, 2 * HL), F32)],
    )(proj.reshape(T, 2 * B, 8 * HL), Whh_f.T, Whh_b.T, Watt, Wcls,
      bcls.reshape(1, NCLS))
    return att, cls


# speedup vs baseline: 7.0597x; 1.0260x over previous
import jax
import jax.numpy as jnp
from jax.experimental import pallas as pl
from jax.experimental.pallas import tpu as pltpu

B, T, V, E = 32, 12, 17, 38
HEADS, HC, HL, NCLS = 8, 128, 512, 500
G = 2 * T
NGV = G * V
NGE = G * E
HID = HEADS * HC
D = HC * (V + 2)
F32 = jnp.float32
HI = jax.lax.Precision.HIGHEST


def _gat_body(x_ref, sels_ref, selt_ref, seltt_ref, w1_ref, as1_ref, ad1_ref,
              b1_ref, w2_ref, as2_ref, ad2_ref, b2_ref, sum8_ref, rep8_ref,
              avg_ref, out_ref):
    def layer(h, asrc, adst):
        als = jnp.dot(h * asrc, sum8_ref[...], preferred_element_type=F32, precision=HI)
        ald = jnp.dot(h * adst, sum8_ref[...], preferred_element_type=F32, precision=HI)
        al = (jnp.dot(sels_ref[...], als, preferred_element_type=F32, precision=HI)
              + jnp.dot(selt_ref[...], ald, preferred_element_type=F32, precision=HI))
        al = jnp.where(al >= 0.0, al, 0.2 * al)
        m = jnp.max(al, axis=0, keepdims=True)
        e = jnp.exp(al - m)
        den = jnp.dot(seltt_ref[...], e, preferred_element_type=F32, precision=HI)
        den_e = jnp.dot(selt_ref[...], den, preferred_element_type=F32, precision=HI)
        a = e / (den_e + 1e-16)
        hs = jnp.dot(sels_ref[...], h, preferred_element_type=F32, precision=HI)
        aexp = jnp.dot(a, rep8_ref[...], preferred_element_type=F32, precision=HI)
        return jnp.dot(seltt_ref[...], hs * aexp, preferred_element_type=F32, precision=HI)

    h1 = jnp.dot(x_ref[...], w1_ref[...], preferred_element_type=F32, precision=HI)
    o1 = layer(h1, as1_ref[...], ad1_ref[...]) + b1_ref[...]
    x1 = jnp.where(o1 > 0.0, o1, jnp.exp(jnp.minimum(o1, 0.0)) - 1.0)
    h2 = jnp.dot(x1, w2_ref[...], preferred_element_type=F32, precision=HI)
    o2 = layer(h2, as2_ref[...], ad2_ref[...])
    out_ref[...] = jnp.dot(o2, avg_ref[...], preferred_element_type=F32, precision=HI) + b2_ref[...]


def _edge_body(px_ref, py_ref, d0_ref, d1_ref, wa_ref, wb_ref, bfe_ref, out_ref):
    px, py = px_ref[...], py_ref[...]
    for r, d_ref in ((0, d0_ref), (1, d1_ref)):
        vx = jnp.dot(px, d_ref[...], preferred_element_type=F32, precision=HI)
        vy = jnp.dot(py, d_ref[...], preferred_element_type=F32, precision=HI)
        ln = jnp.sqrt(vx * vx + vy * vy)
        ang = jnp.arctan2(vy, vx)
        o = (jnp.dot(ln, wa_ref[...], preferred_element_type=F32, precision=HI)
             + jnp.dot(ang, wb_ref[...], preferred_element_type=F32, precision=HI)
             + bfe_ref[...])
        out_ref[:, r * HC:(r + 1) * HC] = o


def _proj_body(xe_ref, scv_ref, shv_ref, xg_ref, scg_ref, shg_ref, sel_ref,
               wte_ref, wtg_ref, b_ref, out_ref):
    dn = (((1,), (1,)), ((), ()))
    xg = xg_ref[...] * scg_ref[...] + shg_ref[...]
    g36 = jax.lax.dot_general(xg, wtg_ref[...], dn, preferred_element_type=F32, precision=HI)
    xe = xe_ref[...] * scv_ref[...] + shv_ref[...]
    out_ref[...] = (jax.lax.dot_general(xe, wte_ref[...], dn, preferred_element_type=F32, precision=HI)
                    + jnp.dot(sel_ref[...], g36, preferred_element_type=F32, precision=HI)
                    + b_ref[0])


def _lstm_body(g_ref, whf_ref, whb_ref, watt_ref, wcls_ref, bcls_ref,
               att_ref, cls_ref, lo_ref):
    nb = 2 * B

    def cell(g):
        i = jax.nn.sigmoid(g[:, 0:HL])
        f = jax.nn.sigmoid(g[:, HL:2 * HL])
        gg = jnp.tanh(g[:, 2 * HL:3 * HL])
        o = jax.nn.sigmoid(g[:, 3 * HL:4 * HL])
        return i, f, gg, o

    hf = jnp.zeros((nb, HL), F32)
    cf = jnp.zeros((nb, HL), F32)
    hb = jnp.zeros((nb, HL), F32)
    cb = jnp.zeros((nb, HL), F32)
    for t in range(T):
        gf = g_ref[t, :, 0:4 * HL] + jax.lax.dot_general(hf, whf_ref[...], (((1,), (1,)), ((), ())), preferred_element_type=F32, precision=HI)
        i, f, gg, o = cell(gf)
        cf = f * cf + i * gg
        hf = o * jnp.tanh(cf)
        lo_ref[t, :, 0:HL] = hf
        tb = T - 1 - t
        gb = g_ref[tb, :, 4 * HL:8 * HL] + jax.lax.dot_general(hb, whb_ref[...], (((1,), (1,)), ((), ())), preferred_element_type=F32, precision=HI)
        i, f, gg, o = cell(gb)
        cb = f * cb + i * gg
        hb = o * jnp.tanh(cb)
        lo_ref[tb, :, HL:2 * HL] = hb

    scores = jnp.concatenate(
        [jnp.dot(lo_ref[t], watt_ref[...], preferred_element_type=F32, precision=HI)
         for t in range(T)], axis=1)
    m = jnp.max(scores, axis=1, keepdims=True)
    e = jnp.exp(scores - m)
    aw = e / jnp.sum(e, axis=1, keepdims=True)
    att = jnp.zeros((nb, 2 * HL), F32)
    for t in range(T):
        att = att + aw[:, t:t + 1] * lo_ref[t]
    att_ref[...] = att
    cls_ref[...] = jnp.dot(att, wcls_ref[...], preferred_element_type=F32, precision=HI) + bcls_ref[...]


def kernel(pose1, pose2, connections, W1, att_src1, att_dst1, b1, W2, att_src2,
           att_dst2, b2, Wfe, bfe, bn_gamma, bn_beta, bn_mean, bn_var, Wih_f,
           Whh_f, bih_f, bhh_f, Wih_b, Whh_b, bih_b, bhh_b, Watt, batt, Wcls,
           bcls):
    s_idx = connections[0].astype(jnp.int32)
    t_idx = connections[1].astype(jnp.int32)

    gofs = jnp.arange(G, dtype=jnp.int32) * V
    sel_s = jax.nn.one_hot((gofs[:, None] + s_idx[None, :]).reshape(NGE), NGV, dtype=F32)
    sel_t = jax.nn.one_hot((gofs[:, None] + t_idx[None, :]).reshape(NGE), NGV, dtype=F32)
    sel_tt = sel_t.T
    sum8 = jax.nn.one_hot(jnp.arange(HID, dtype=jnp.int32) // HC, HEADS, dtype=F32)
    rep8 = sum8.T
    avg8 = jax.nn.one_hot(jnp.arange(HID, dtype=jnp.int32) % HC, HC, dtype=F32) / HEADS

    x24 = jnp.concatenate([pose1[0], pose2[0]], axis=0).reshape(NGV, 3)
    gat_nodes = pl.pallas_call(
        _gat_body,
        out_shape=jax.ShapeDtypeStruct((NGV, HC), F32),
    )(x24, sel_s, sel_t, sel_tt, W1, att_src1.reshape(1, HID),
      att_dst1.reshape(1, HID), b1.reshape(1, HID), W2,
      att_src2.reshape(1, HID), att_dst2.reshape(1, HID), b2.reshape(1, HC),
      sum8, rep8, avg8)

    pall = jnp.concatenate([pose1, pose2], axis=0).transpose(1, 0, 2, 3)
    pall = pall.reshape(2 * B * T, V, 3)
    px, py = pall[:, :, 0], pall[:, :, 1]
    dmat = (jax.nn.one_hot(t_idx, V, dtype=F32) - jax.nn.one_hot(s_idx, V, dtype=F32)).T
    edge_out = pl.pallas_call(
        _edge_body,
        out_shape=jax.ShapeDtypeStruct((2 * B * T, 2 * HC), F32),
    )(px, py, dmat[:, :E // 2], dmat[:, E // 2:], Wfe[0::2], Wfe[1::2],
      bfe.reshape(1, HC))

    sc = bn_gamma / jnp.sqrt(bn_var + 1e-5)
    sh = bn_beta - bn_mean * sc
    gat2 = gat_nodes.reshape(G, V * HC)
    dead = jnp.tile(b2, V)
    xg36 = jnp.concatenate(
        [jnp.broadcast_to(dead, (T, V * HC)), gat2], axis=0)
    scg = jnp.tile(sc, 3).reshape(3 * T, 1)
    shg = jnp.tile(sh, 3).reshape(3 * T, 1)
    tcol = jnp.arange(2 * B * T, dtype=jnp.int32) // (2 * B)
    bcol = jnp.arange(2 * B * T, dtype=jnp.int32) % (2 * B)
    sel_idx = jnp.where(bcol == 0, 12 + tcol,
                        jnp.where(bcol == B, 12 + T + tcol, tcol))
    sel768 = jax.nn.one_hot(sel_idx, 3 * T, dtype=F32)

    scv = jnp.repeat(sc, 2 * B).reshape(2 * B * T, 1)
    shv = jnp.repeat(sh, 2 * B).reshape(2 * B * T, 1)
    wt = jnp.concatenate([Wih_f, Wih_b], axis=0)
    wte = wt[:, V * HC:]
    wtg = wt[:, :V * HC]
    bias = jnp.concatenate([bih_f + bhh_f, bih_b + bhh_b]).reshape(8, 1, HL)
    nblk = 8
    proj = pl.pallas_call(
        _proj_body,
        grid=(nblk,),
        in_specs=[
            pl.BlockSpec((2 * B * T, 2 * HC), lambda i: (0, 0)),
            pl.BlockSpec((2 * B * T, 1), lambda i: (0, 0)),
            pl.BlockSpec((2 * B * T, 1), lambda i: (0, 0)),
            pl.BlockSpec((3 * T, V * HC), lambda i: (0, 0)),
            pl.BlockSpec((3 * T, 1), lambda i: (0, 0)),
            pl.BlockSpec((3 * T, 1), lambda i: (0, 0)),
            pl.BlockSpec((2 * B * T, 3 * T), lambda i: (0, 0)),
            pl.BlockSpec((HL, 2 * HC), lambda i: (i, 0)),
            pl.BlockSpec((HL, V * HC), lambda i: (i, 0)),
            pl.BlockSpec((1, 1, HL), lambda i: (i, 0, 0)),
        ],
        out_specs=pl.BlockSpec((2 * B * T, HL), lambda i: (0, i)),
        out_shape=jax.ShapeDtypeStruct((2 * B * T, 8 * HL), F32),
    )(edge_out, scv, shv, xg36, scg, shg, sel768, wte, wtg, bias)

    att, cls = pl.pallas_call(
        _lstm_body,
        out_shape=(jax.ShapeDtypeStruct((2 * B, 2 * HL), F32),
                   jax.ShapeDtypeStruct((2 * B, NCLS), F32)),
        scratch_shapes=[pltpu.VMEM((T, 2 * B, 2 * HL), F32)],
    )(proj.reshape(T, 2 * B, 8 * HL), Whh_f, Whh_b, Watt, Wcls,
      bcls.reshape(1, NCLS))
    return att, cls
